# baseline (speedup 1.0000x reference)
import functools

import jax
import jax.numpy as jnp
from jax import lax
from jax.experimental import pallas as pl
from jax.experimental.pallas import tpu as pltpu
from jax.experimental.pallas import tpu_sc as plsc

D = 768
E = 8
DFF = 2 * D
N = 2048
A = 2 * N
T = 512
NTILES = A // T
U = NTILES + E - 1

NC = 2
NS = 16
NW = NC * NS
BPW = A // NW

TNL = 256


def _route_body(x_ref, wg_ref, bg_ref, idx_ref, w_ref):
    logits = lax.dot_general(
        x_ref[...], wg_ref[...],
        dimension_numbers=(((1,), (1,)), ((), ())),
        preferred_element_type=jnp.float32,
    ) + bg_ref[...]
    ii = lax.broadcasted_iota(jnp.int32, (N, E), 1)
    v1 = jnp.max(logits, axis=1, keepdims=True)
    i1 = jnp.min(jnp.where(logits == v1, ii, E), axis=1, keepdims=True)
    m1 = ii == i1
    neg = jnp.where(m1, -jnp.inf, logits)
    v2 = jnp.max(neg, axis=1, keepdims=True)
    i2 = jnp.min(jnp.where(neg == v2, ii, E), axis=1, keepdims=True)
    z = jnp.exp(v2 - v1)
    sm1 = 1.0 / (1.0 + z)
    sm2 = z / (1.0 + z)
    idx_ref[...] = jnp.concatenate([i1, i2], axis=1)
    w_ref[...] = jnp.concatenate([sm1, sm2], axis=1)


def _route(x, Wg, bg2):
    return pl.pallas_call(
        _route_body,
        out_shape=(jax.ShapeDtypeStruct((N, 2), jnp.int32),
                   jax.ShapeDtypeStruct((N, 2), jnp.float32)),
    )(x, Wg, bg2)


def _sc_reorder(x, tokidx, dest):
    @functools.partial(
        pl.kernel,
        out_type=jax.ShapeDtypeStruct((A, D), jnp.float32),
        mesh=plsc.VectorSubcoreMesh(core_axis_name="c", subcore_axis_name="s"),
        scratch_types=[
            pltpu.VMEM((BPW,), jnp.int32),
            pltpu.VMEM((BPW,), jnp.int32),
            pltpu.VMEM((BPW, D), jnp.float32),
            pltpu.SemaphoreType.DMA,
        ],
    )
    def k(x_hbm, tok_hbm, dst_hbm, out_hbm, tok_v, dst_v, rows_v, sem):
        wid = lax.axis_index("s") * NC + lax.axis_index("c")
        base = wid * BPW
        pltpu.sync_copy(tok_hbm.at[pl.ds(base, BPW)], tok_v)
        pltpu.sync_copy(dst_hbm.at[pl.ds(base, BPW)], dst_v)
        pltpu.async_copy(x_hbm.at[tok_v], rows_v, sem).wait()
        pltpu.async_copy(rows_v, out_hbm.at[dst_v], sem).wait()

    return k(x, tokidx, dest)


def _sc_unsort(ys, g):
    @functools.partial(
        pl.kernel,
        out_type=jax.ShapeDtypeStruct((A, D), jnp.float32),
        mesh=plsc.VectorSubcoreMesh(core_axis_name="c", subcore_axis_name="s"),
        scratch_types=[
            pltpu.VMEM((BPW,), jnp.int32),
            pltpu.VMEM((BPW, D), jnp.float32),
            pltpu.SemaphoreType.DMA,
        ],
    )
    def k(ys_hbm, g_hbm, out_hbm, idx_v, rows_v, sem):
        wid = lax.axis_index("s") * NC + lax.axis_index("c")
        base = wid * BPW
        pltpu.sync_copy(g_hbm.at[pl.ds(base, BPW)], idx_v)
        pltpu.async_copy(ys_hbm.at[idx_v], rows_v, sem).wait()
        pltpu.sync_copy(rows_v, out_hbm.at[pl.ds(base, BPW)])

    return k(ys, g)


def _ffn_body(utile_s, uexp_s, ulo_s, uhi_s,
              xs_ref, w1_ref, b1_ref, w2_ref, b2_ref, ys_ref):
    u = pl.program_id(0)
    lo = ulo_s[u]
    hi = uhi_s[u]
    t = utile_s[u]

    @pl.when(hi > lo)
    def _():
        h = lax.dot_general(
            xs_ref[...].astype(jnp.bfloat16), w1_ref[0],
            dimension_numbers=(((1,), (1,)), ((), ())),
            preferred_element_type=jnp.float32,
        ) + b1_ref[0]
        h = jnp.maximum(h, 0.0)
        y = lax.dot_general(
            h.astype(jnp.bfloat16), w2_ref[0],
            dimension_numbers=(((1,), (1,)), ((), ())),
            preferred_element_type=jnp.float32,
        ) + b2_ref[0]
        rows = lax.broadcasted_iota(jnp.int32, (T, 1), 0) + t * T
        mask = jnp.logical_and(rows >= lo, rows < hi)
        ys_ref[...] = jnp.where(mask, y, ys_ref[...])


def _ffn(xs, W1, b1, W2, b2, utile, uexp, ulo, uhi):
    grid_spec = pltpu.PrefetchScalarGridSpec(
        num_scalar_prefetch=4,
        grid=(U,),
        in_specs=[
            pl.BlockSpec((T, D), lambda u, ut, ue, ul, uh: (ut[u], 0)),
            pl.BlockSpec((1, DFF, D), lambda u, ut, ue, ul, uh: (ue[u], 0, 0)),
            pl.BlockSpec((1, 1, DFF), lambda u, ut, ue, ul, uh: (ue[u], 0, 0)),
            pl.BlockSpec((1, D, DFF), lambda u, ut, ue, ul, uh: (ue[u], 0, 0)),
            pl.BlockSpec((1, 1, D), lambda u, ut, ue, ul, uh: (ue[u], 0, 0)),
        ],
        out_specs=pl.BlockSpec((T, D), lambda u, ut, ue, ul, uh: (ut[u], 0)),
    )
    return pl.pallas_call(
        _ffn_body,
        grid_spec=grid_spec,
        out_shape=jax.ShapeDtypeStruct((A, D), jnp.float32),
    )(utile, uexp, ulo, uhi, xs, W1, b1, W2, b2)


def _ln_body(x_ref, tw_ref, b0_ref, b1_ref, gamma_ref, beta_ref, out_ref):
    w0 = tw_ref[:, 0:1]
    w1 = tw_ref[:, 1:2]
    res = x_ref[...] + w0 * b0_ref[...] + w1 * b1_ref[...]
    mu = jnp.mean(res, axis=1, keepdims=True)
    var = jnp.mean((res - mu) ** 2, axis=1, keepdims=True)
    out_ref[...] = (gamma_ref[...] * (res - mu)
                    * lax.rsqrt(var + 1e-5) + beta_ref[...])


def _ln(x, top_w, buf, gamma2, beta2):
    nt = N // TNL
    return pl.pallas_call(
        _ln_body,
        grid=(nt,),
        in_specs=[
            pl.BlockSpec((TNL, D), lambda t: (t, 0)),
            pl.BlockSpec((TNL, 2), lambda t: (t, 0)),
            pl.BlockSpec((TNL, D), lambda t: (t, 0)),
            pl.BlockSpec((TNL, D), lambda t: (t + N // TNL, 0)),
            pl.BlockSpec((1, D), lambda t: (0, 0)),
            pl.BlockSpec((1, D), lambda t: (0, 0)),
        ],
        out_specs=pl.BlockSpec((TNL, D), lambda t: (t, 0)),
        out_shape=jax.ShapeDtypeStruct((N, D), jnp.float32),
    )(x, top_w, buf, buf, gamma2, beta2)


@jax.jit
def kernel(x, Wg, bg, W1, b1, W2, b2, gamma, beta):
    i32 = jnp.int32
    bg2 = bg.reshape(1, E)
    gamma2 = gamma.reshape(1, D)
    beta2 = beta.reshape(1, D)

    top_idx, top_w = _route(x, Wg, bg2)

    ef = top_idx.reshape(-1)
    oh = (ef[:, None] == jnp.arange(E, dtype=i32)[None, :]).astype(i32)
    ranks = jnp.cumsum(oh, axis=0)
    counts = ranks[-1]
    ends = jnp.cumsum(counts)
    offsets = ends - counts
    rank_a = jnp.sum(oh * ranks, axis=1)
    base_a = jnp.sum(oh * offsets[None, :], axis=1)
    dest = (base_a + rank_a - 1).astype(i32)
    tokidx = (jnp.arange(A, dtype=i32) // 2)
    g = dest.reshape(N, 2).T.reshape(-1)

    starts_r = jnp.arange(NTILES, dtype=i32) * T
    e_first = jnp.searchsorted(ends, starts_r, side="right").astype(i32)
    e_last = jnp.searchsorted(ends, starts_r + (T - 1), side="right").astype(i32)
    spans = e_last - e_first
    cum_units = jnp.arange(NTILES, dtype=i32) + jnp.concatenate(
        [jnp.zeros((1,), i32), jnp.cumsum(spans)[:-1]])
    ss = jnp.arange(U, dtype=i32)
    ut = (jnp.searchsorted(cum_units, ss, side="right").astype(i32) - 1)
    oht = (ut[:, None] == jnp.arange(NTILES, dtype=i32)[None, :]).astype(i32)
    jj = ss - jnp.sum(oht * cum_units[None, :], axis=1)
    ue_raw = jnp.sum(oht * e_first[None, :], axis=1) + jj
    valid = ue_raw <= jnp.sum(oht * e_last[None, :], axis=1)
    ue_c = jnp.clip(ue_raw, 0, E - 1)
    ohe = (ue_c[:, None] == jnp.arange(E, dtype=i32)[None, :]).astype(i32)
    lo = jnp.maximum(jnp.sum(ohe * offsets[None, :], axis=1), ut * T)
    hi = jnp.minimum(jnp.sum(ohe * ends[None, :], axis=1), (ut + 1) * T)
    ulo = jnp.where(valid, lo, 0).astype(i32)
    uhi = jnp.where(valid, hi, 0).astype(i32)
    ue = jnp.maximum(
        lax.cummax(jnp.where(uhi > ulo, ue_c, -1)), 0).astype(i32)

    xs = _sc_reorder(x, tokidx, dest)
    ys = _ffn(xs, W1.astype(jnp.bfloat16), b1.reshape(E, 1, DFF),
              W2.astype(jnp.bfloat16), b2.reshape(E, 1, D),
              ut, ue, ulo, uhi)
    buf = _sc_unsort(ys, g)
    return _ln(x, top_w, buf, gamma2, beta2)


# speedup vs baseline: 1.2065x; 1.2065x over previous
import functools

import jax
import jax.numpy as jnp
from jax import lax
from jax.experimental import pallas as pl
from jax.experimental.pallas import tpu as pltpu
from jax.experimental.pallas import tpu_sc as plsc

D = 768
E = 8
DFF = 2 * D
N = 2048
A = 2 * N
T = 256
P = A + E * T
NTP = P // T

NC = 2
NS = 16
NW = NC * NS
BPW = A // NW

TNL = 256


def _route_body(x_ref, wg_ref, bg_ref, idx_ref, w_ref):
    logits = lax.dot_general(
        x_ref[...], wg_ref[...],
        dimension_numbers=(((1,), (1,)), ((), ())),
        preferred_element_type=jnp.float32,
    ) + bg_ref[...]
    ii = lax.broadcasted_iota(jnp.int32, (N, E), 1)
    v1 = jnp.max(logits, axis=1, keepdims=True)
    i1 = jnp.min(jnp.where(logits == v1, ii, E), axis=1, keepdims=True)
    m1 = ii == i1
    neg = jnp.where(m1, -jnp.inf, logits)
    v2 = jnp.max(neg, axis=1, keepdims=True)
    i2 = jnp.min(jnp.where(neg == v2, ii, E), axis=1, keepdims=True)
    z = jnp.exp(v2 - v1)
    sm1 = 1.0 / (1.0 + z)
    sm2 = z / (1.0 + z)
    idx_ref[...] = jnp.concatenate([i1, i2], axis=1)
    w_ref[...] = jnp.concatenate([sm1, sm2], axis=1)


def _route(x, Wg, bg2):
    return pl.pallas_call(
        _route_body,
        out_shape=(jax.ShapeDtypeStruct((N, 2), jnp.int32),
                   jax.ShapeDtypeStruct((N, 2), jnp.float32)),
    )(x, Wg, bg2)


def _sc_reorder(x, tokidx, dest):
    @functools.partial(
        pl.kernel,
        out_type=jax.ShapeDtypeStruct((P, D), jnp.float32),
        mesh=plsc.VectorSubcoreMesh(core_axis_name="c", subcore_axis_name="s"),
        scratch_types=[
            pltpu.VMEM((BPW,), jnp.int32),
            pltpu.VMEM((BPW,), jnp.int32),
            pltpu.VMEM((BPW, D), jnp.float32),
            pltpu.SemaphoreType.DMA,
        ],
    )
    def k(x_hbm, tok_hbm, dst_hbm, out_hbm, tok_v, dst_v, rows_v, sem):
        wid = lax.axis_index("s") * NC + lax.axis_index("c")
        base = wid * BPW
        pltpu.sync_copy(tok_hbm.at[pl.ds(base, BPW)], tok_v)
        pltpu.sync_copy(dst_hbm.at[pl.ds(base, BPW)], dst_v)
        pltpu.async_copy(x_hbm.at[tok_v], rows_v, sem).wait()
        pltpu.async_copy(rows_v, out_hbm.at[dst_v], sem).wait()

    return k(x, tokidx, dest)


def _sc_unsort(ys, g):
    @functools.partial(
        pl.kernel,
        out_type=jax.ShapeDtypeStruct((A, D), jnp.float32),
        mesh=plsc.VectorSubcoreMesh(core_axis_name="c", subcore_axis_name="s"),
        scratch_types=[
            pltpu.VMEM((BPW,), jnp.int32),
            pltpu.VMEM((BPW, D), jnp.float32),
            pltpu.SemaphoreType.DMA,
        ],
    )
    def k(ys_hbm, g_hbm, out_hbm, idx_v, rows_v, sem):
        wid = lax.axis_index("s") * NC + lax.axis_index("c")
        base = wid * BPW
        pltpu.sync_copy(g_hbm.at[pl.ds(base, BPW)], idx_v)
        pltpu.async_copy(ys_hbm.at[idx_v], rows_v, sem).wait()
        pltpu.sync_copy(rows_v, out_hbm.at[pl.ds(base, BPW)])

    return k(ys, g)


def _ffn_body(uexp_s, uact_s, xs_ref, w1_ref, b1_ref, w2_ref, b2_ref,
              ys_ref, w1c_ref, w2c_ref):
    t = pl.program_id(0)
    changed = jnp.logical_or(
        t == 0, uexp_s[t] != uexp_s[jnp.maximum(t - 1, 0)])

    @pl.when(jnp.logical_and(changed, uact_s[t] > 0))
    def _cast():
        w1c_ref[...] = w1_ref[0].astype(jnp.bfloat16)
        w2c_ref[...] = w2_ref[0].astype(jnp.bfloat16)

    @pl.when(uact_s[t] > 0)
    def _():
        h = lax.dot_general(
            xs_ref[...].astype(jnp.bfloat16), w1c_ref[...],
            dimension_numbers=(((1,), (1,)), ((), ())),
            preferred_element_type=jnp.float32,
        ) + b1_ref[0]
        h = jnp.maximum(h, 0.0)
        ys_ref[...] = lax.dot_general(
            h.astype(jnp.bfloat16), w2c_ref[...],
            dimension_numbers=(((1,), (1,)), ((), ())),
            preferred_element_type=jnp.float32,
        ) + b2_ref[0]


def _ffn(xs, W1, b1, W2, b2, uexp, uact):
    grid_spec = pltpu.PrefetchScalarGridSpec(
        num_scalar_prefetch=2,
        grid=(NTP,),
        in_specs=[
            pl.BlockSpec((T, D), lambda t, ue, ua: (t, 0)),
            pl.BlockSpec((1, DFF, D), lambda t, ue, ua: (ue[t], 0, 0)),
            pl.BlockSpec((1, 1, DFF), lambda t, ue, ua: (ue[t], 0, 0)),
            pl.BlockSpec((1, D, DFF), lambda t, ue, ua: (ue[t], 0, 0)),
            pl.BlockSpec((1, 1, D), lambda t, ue, ua: (ue[t], 0, 0)),
        ],
        out_specs=pl.BlockSpec((T, D), lambda t, ue, ua: (t, 0)),
        scratch_shapes=[
            pltpu.VMEM((DFF, D), jnp.bfloat16),
            pltpu.VMEM((D, DFF), jnp.bfloat16),
        ],
    )
    return pl.pallas_call(
        _ffn_body,
        grid_spec=grid_spec,
        out_shape=jax.ShapeDtypeStruct((P, D), jnp.float32),
    )(uexp, uact, xs, W1, b1, W2, b2)


def _ln_body(x_ref, tw_ref, b0_ref, b1_ref, gamma_ref, beta_ref, out_ref):
    w0 = tw_ref[:, 0:1]
    w1 = tw_ref[:, 1:2]
    res = x_ref[...] + w0 * b0_ref[...] + w1 * b1_ref[...]
    mu = jnp.mean(res, axis=1, keepdims=True)
    var = jnp.mean((res - mu) ** 2, axis=1, keepdims=True)
    out_ref[...] = (gamma_ref[...] * (res - mu)
                    * lax.rsqrt(var + 1e-5) + beta_ref[...])


def _ln(x, top_w, buf, gamma2, beta2):
    nt = N // TNL
    return pl.pallas_call(
        _ln_body,
        grid=(nt,),
        in_specs=[
            pl.BlockSpec((TNL, D), lambda t: (t, 0)),
            pl.BlockSpec((TNL, 2), lambda t: (t, 0)),
            pl.BlockSpec((TNL, D), lambda t: (t, 0)),
            pl.BlockSpec((TNL, D), lambda t: (t + N // TNL, 0)),
            pl.BlockSpec((1, D), lambda t: (0, 0)),
            pl.BlockSpec((1, D), lambda t: (0, 0)),
        ],
        out_specs=pl.BlockSpec((TNL, D), lambda t: (t, 0)),
        out_shape=jax.ShapeDtypeStruct((N, D), jnp.float32),
    )(x, top_w, buf, buf, gamma2, beta2)


@jax.jit
def kernel(x, Wg, bg, W1, b1, W2, b2, gamma, beta):
    i32 = jnp.int32
    bg2 = bg.reshape(1, E)
    gamma2 = gamma.reshape(1, D)
    beta2 = beta.reshape(1, D)

    top_idx, top_w = _route(x, Wg, bg2)

    ef = top_idx.reshape(-1)
    oh = (ef[:, None] == jnp.arange(E, dtype=i32)[None, :]).astype(i32)
    ranks = jnp.cumsum(oh, axis=0)
    counts = ranks[-1]
    pcount = ((counts + (T - 1)) // T) * T
    pends = jnp.cumsum(pcount)
    poff = pends - pcount
    rank_a = jnp.sum(oh * ranks, axis=1)
    base_a = jnp.sum(oh * poff[None, :], axis=1)
    dest = (base_a + rank_a - 1).astype(i32)
    tokidx = (jnp.arange(A, dtype=i32) // 2)
    g = dest.reshape(N, 2).T.reshape(-1)

    starts_r = jnp.arange(NTP, dtype=i32) * T
    ue_raw = jnp.searchsorted(pends, starts_r, side="right").astype(i32)
    uact = (starts_r < pends[-1]).astype(i32)
    ue = jnp.maximum(
        lax.cummax(jnp.where(uact > 0, jnp.clip(ue_raw, 0, E - 1), -1)),
        0).astype(i32)

    xs = _sc_reorder(x, tokidx, dest)
    ys = _ffn(xs, W1, b1.reshape(E, 1, DFF), W2, b2.reshape(E, 1, D),
              ue, uact)
    buf = _sc_unsort(ys, g)
    return _ln(x, top_w, buf, gamma2, beta2)


# speedup vs baseline: 1.2876x; 1.0673x over previous
import functools

import jax
import jax.numpy as jnp
from jax import lax
from jax.experimental import pallas as pl
from jax.experimental.pallas import tpu as pltpu
from jax.experimental.pallas import tpu_sc as plsc

D = 768
E = 8
DFF = 2 * D
N = 2048
A = 2 * N
T = 512
P = A + E * T
NTP = P // T

NC = 2
NS = 16
NW = NC * NS
BPW = A // NW

TNL = 256


def _route_body(x_ref, wg_ref, bg_ref, idx_ref, w_ref):
    logits = lax.dot_general(
        x_ref[...], wg_ref[...],
        dimension_numbers=(((1,), (1,)), ((), ())),
        preferred_element_type=jnp.float32,
    ) + bg_ref[...]
    ii = lax.broadcasted_iota(jnp.int32, (N, E), 1)
    v1 = jnp.max(logits, axis=1, keepdims=True)
    i1 = jnp.min(jnp.where(logits == v1, ii, E), axis=1, keepdims=True)
    m1 = ii == i1
    neg = jnp.where(m1, -jnp.inf, logits)
    v2 = jnp.max(neg, axis=1, keepdims=True)
    i2 = jnp.min(jnp.where(neg == v2, ii, E), axis=1, keepdims=True)
    z = jnp.exp(v2 - v1)
    sm1 = 1.0 / (1.0 + z)
    sm2 = z / (1.0 + z)
    idx_ref[...] = jnp.concatenate([i1, i2], axis=1)
    w_ref[...] = jnp.concatenate([sm1, sm2], axis=1)


def _route(x, Wg, bg2):
    return pl.pallas_call(
        _route_body,
        out_shape=(jax.ShapeDtypeStruct((N, 2), jnp.int32),
                   jax.ShapeDtypeStruct((N, 2), jnp.float32)),
    )(x, Wg, bg2)


def _sc_reorder(x, tokidx, dest):
    @functools.partial(
        pl.kernel,
        out_type=jax.ShapeDtypeStruct((P, D), jnp.float32),
        mesh=plsc.VectorSubcoreMesh(core_axis_name="c", subcore_axis_name="s"),
        scratch_types=[
            pltpu.VMEM((BPW,), jnp.int32),
            pltpu.VMEM((BPW,), jnp.int32),
            pltpu.VMEM((BPW, D), jnp.float32),
            pltpu.SemaphoreType.DMA,
        ],
    )
    def k(x_hbm, tok_hbm, dst_hbm, out_hbm, tok_v, dst_v, rows_v, sem):
        wid = lax.axis_index("s") * NC + lax.axis_index("c")
        base = wid * BPW
        pltpu.sync_copy(tok_hbm.at[pl.ds(base, BPW)], tok_v)
        pltpu.sync_copy(dst_hbm.at[pl.ds(base, BPW)], dst_v)
        pltpu.async_copy(x_hbm.at[tok_v], rows_v, sem).wait()
        pltpu.async_copy(rows_v, out_hbm.at[dst_v], sem).wait()

    return k(x, tokidx, dest)


def _sc_unsort(ys, g):
    @functools.partial(
        pl.kernel,
        out_type=jax.ShapeDtypeStruct((A, D), jnp.float32),
        mesh=plsc.VectorSubcoreMesh(core_axis_name="c", subcore_axis_name="s"),
        scratch_types=[
            pltpu.VMEM((BPW,), jnp.int32),
            pltpu.VMEM((BPW, D), jnp.float32),
            pltpu.SemaphoreType.DMA,
        ],
    )
    def k(ys_hbm, g_hbm, out_hbm, idx_v, rows_v, sem):
        wid = lax.axis_index("s") * NC + lax.axis_index("c")
        base = wid * BPW
        pltpu.sync_copy(g_hbm.at[pl.ds(base, BPW)], idx_v)
        pltpu.async_copy(ys_hbm.at[idx_v], rows_v, sem).wait()
        pltpu.sync_copy(rows_v, out_hbm.at[pl.ds(base, BPW)])

    return k(ys, g)


def _ffn_body(uexp_s, uact_s, xs_ref, w1_ref, b1_ref, w2_ref, b2_ref,
              ys_ref, w1c_ref, w2c_ref):
    t = pl.program_id(0)
    changed = jnp.logical_or(
        t == 0, uexp_s[t] != uexp_s[jnp.maximum(t - 1, 0)])

    @pl.when(jnp.logical_and(changed, uact_s[t] > 0))
    def _cast():
        w1c_ref[...] = w1_ref[0].astype(jnp.bfloat16)
        w2c_ref[...] = w2_ref[0].astype(jnp.bfloat16)

    @pl.when(uact_s[t] > 0)
    def _():
        h = lax.dot_general(
            xs_ref[...].astype(jnp.bfloat16), w1c_ref[...],
            dimension_numbers=(((1,), (1,)), ((), ())),
            preferred_element_type=jnp.float32,
        ) + b1_ref[0]
        h = jnp.maximum(h, 0.0)
        ys_ref[...] = lax.dot_general(
            h.astype(jnp.bfloat16), w2c_ref[...],
            dimension_numbers=(((1,), (1,)), ((), ())),
            preferred_element_type=jnp.float32,
        ) + b2_ref[0]


def _ffn(xs, W1, b1, W2, b2, uexp, uact):
    grid_spec = pltpu.PrefetchScalarGridSpec(
        num_scalar_prefetch=2,
        grid=(NTP,),
        in_specs=[
            pl.BlockSpec((T, D), lambda t, ue, ua: (t, 0)),
            pl.BlockSpec((1, DFF, D), lambda t, ue, ua: (ue[t], 0, 0)),
            pl.BlockSpec((1, 1, DFF), lambda t, ue, ua: (ue[t], 0, 0)),
            pl.BlockSpec((1, D, DFF), lambda t, ue, ua: (ue[t], 0, 0)),
            pl.BlockSpec((1, 1, D), lambda t, ue, ua: (ue[t], 0, 0)),
        ],
        out_specs=pl.BlockSpec((T, D), lambda t, ue, ua: (t, 0)),
        scratch_shapes=[
            pltpu.VMEM((DFF, D), jnp.bfloat16),
            pltpu.VMEM((D, DFF), jnp.bfloat16),
        ],
    )
    return pl.pallas_call(
        _ffn_body,
        grid_spec=grid_spec,
        out_shape=jax.ShapeDtypeStruct((P, D), jnp.float32),
    )(uexp, uact, xs, W1, b1, W2, b2)


def _ln_body(x_ref, tw_ref, b0_ref, b1_ref, gamma_ref, beta_ref, out_ref):
    w0 = tw_ref[:, 0:1]
    w1 = tw_ref[:, 1:2]
    res = x_ref[...] + w0 * b0_ref[...] + w1 * b1_ref[...]
    mu = jnp.mean(res, axis=1, keepdims=True)
    var = jnp.mean((res - mu) ** 2, axis=1, keepdims=True)
    out_ref[...] = (gamma_ref[...] * (res - mu)
                    * lax.rsqrt(var + 1e-5) + beta_ref[...])


def _ln(x, top_w, buf, gamma2, beta2):
    nt = N // TNL
    return pl.pallas_call(
        _ln_body,
        grid=(nt,),
        in_specs=[
            pl.BlockSpec((TNL, D), lambda t: (t, 0)),
            pl.BlockSpec((TNL, 2), lambda t: (t, 0)),
            pl.BlockSpec((TNL, D), lambda t: (t, 0)),
            pl.BlockSpec((TNL, D), lambda t: (t + N // TNL, 0)),
            pl.BlockSpec((1, D), lambda t: (0, 0)),
            pl.BlockSpec((1, D), lambda t: (0, 0)),
        ],
        out_specs=pl.BlockSpec((TNL, D), lambda t: (t, 0)),
        out_shape=jax.ShapeDtypeStruct((N, D), jnp.float32),
    )(x, top_w, buf, buf, gamma2, beta2)


@jax.jit
def kernel(x, Wg, bg, W1, b1, W2, b2, gamma, beta):
    i32 = jnp.int32
    bg2 = bg.reshape(1, E)
    gamma2 = gamma.reshape(1, D)
    beta2 = beta.reshape(1, D)

    top_idx, top_w = _route(x, Wg, bg2)

    ef = top_idx.reshape(-1)
    oh = (ef[:, None] == jnp.arange(E, dtype=i32)[None, :]).astype(i32)
    ranks = jnp.cumsum(oh, axis=0)
    counts = ranks[-1]
    pcount = ((counts + (T - 1)) // T) * T
    pends = jnp.cumsum(pcount)
    poff = pends - pcount
    rank_a = jnp.sum(oh * ranks, axis=1)
    base_a = jnp.sum(oh * poff[None, :], axis=1)
    dest = (base_a + rank_a - 1).astype(i32)
    tokidx = (jnp.arange(A, dtype=i32) // 2)
    g = dest.reshape(N, 2).T.reshape(-1)

    starts_r = jnp.arange(NTP, dtype=i32) * T
    ue_raw = jnp.searchsorted(pends, starts_r, side="right").astype(i32)
    uact = (starts_r < pends[-1]).astype(i32)
    ue = jnp.maximum(
        lax.cummax(jnp.where(uact > 0, jnp.clip(ue_raw, 0, E - 1), -1)),
        0).astype(i32)

    xs = _sc_reorder(x, tokidx, dest)
    ys = _ffn(xs, W1, b1.reshape(E, 1, DFF), W2, b2.reshape(E, 1, D),
              ue, uact)
    buf = _sc_unsort(ys, g)
    return _ln(x, top_w, buf, gamma2, beta2)


# speedup vs baseline: 1.2943x; 1.0052x over previous
import functools

import jax
import jax.numpy as jnp
from jax import lax
from jax.experimental import pallas as pl
from jax.experimental.pallas import tpu as pltpu
from jax.experimental.pallas import tpu_sc as plsc

D = 768
E = 8
DFF = 2 * D
N = 2048
A = 2 * N
T = 512
P = A + E * T
NTP = P // T

NC = 2
NS = 16
NW = NC * NS
BPW = A // NW

TNL = 256


def _route_body(x_ref, wg_ref, bg_ref, idx_ref, w_ref):
    logits = lax.dot_general(
        x_ref[...], wg_ref[...],
        dimension_numbers=(((1,), (1,)), ((), ())),
        preferred_element_type=jnp.float32,
    ) + bg_ref[...]
    ii = lax.broadcasted_iota(jnp.int32, (N, E), 1)
    v1 = jnp.max(logits, axis=1, keepdims=True)
    i1 = jnp.min(jnp.where(logits == v1, ii, E), axis=1, keepdims=True)
    m1 = ii == i1
    neg = jnp.where(m1, -jnp.inf, logits)
    v2 = jnp.max(neg, axis=1, keepdims=True)
    i2 = jnp.min(jnp.where(neg == v2, ii, E), axis=1, keepdims=True)
    z = jnp.exp(v2 - v1)
    sm1 = 1.0 / (1.0 + z)
    sm2 = z / (1.0 + z)
    idx_ref[...] = jnp.concatenate([i1, i2], axis=1)
    w_ref[...] = jnp.concatenate([sm1, sm2], axis=1)


def _route(x, Wg, bg2):
    return pl.pallas_call(
        _route_body,
        out_shape=(jax.ShapeDtypeStruct((N, 2), jnp.int32),
                   jax.ShapeDtypeStruct((N, 2), jnp.float32)),
    )(x, Wg, bg2)


def _sc_reorder(x, tokidx, dest):
    @functools.partial(
        pl.kernel,
        out_type=jax.ShapeDtypeStruct((P, D), jnp.float32),
        mesh=plsc.VectorSubcoreMesh(core_axis_name="c", subcore_axis_name="s"),
        scratch_types=[
            pltpu.VMEM((BPW,), jnp.int32),
            pltpu.VMEM((BPW,), jnp.int32),
            pltpu.VMEM((BPW, D), jnp.float32),
            pltpu.SemaphoreType.DMA,
        ],
    )
    def k(x_hbm, tok_hbm, dst_hbm, out_hbm, tok_v, dst_v, rows_v, sem):
        wid = lax.axis_index("s") * NC + lax.axis_index("c")
        base = wid * BPW
        pltpu.sync_copy(tok_hbm.at[pl.ds(base, BPW)], tok_v)
        pltpu.sync_copy(dst_hbm.at[pl.ds(base, BPW)], dst_v)
        pltpu.async_copy(x_hbm.at[tok_v], rows_v, sem).wait()
        pltpu.async_copy(rows_v, out_hbm.at[dst_v], sem).wait()

    return k(x, tokidx, dest)


def _sc_unsort(ys, g):
    @functools.partial(
        pl.kernel,
        out_type=jax.ShapeDtypeStruct((A, D), jnp.float32),
        mesh=plsc.VectorSubcoreMesh(core_axis_name="c", subcore_axis_name="s"),
        scratch_types=[
            pltpu.VMEM((BPW,), jnp.int32),
            pltpu.VMEM((BPW, D), jnp.float32),
            pltpu.SemaphoreType.DMA,
        ],
    )
    def k(ys_hbm, g_hbm, out_hbm, idx_v, rows_v, sem):
        wid = lax.axis_index("s") * NC + lax.axis_index("c")
        base = wid * BPW
        pltpu.sync_copy(g_hbm.at[pl.ds(base, BPW)], idx_v)
        pltpu.async_copy(ys_hbm.at[idx_v], rows_v, sem).wait()
        pltpu.sync_copy(rows_v, out_hbm.at[pl.ds(base, BPW)])

    return k(ys, g)


def _ffn_body(uexp_s, uact_s, xs_ref, w1_ref, b1_ref, w2_ref, b2_ref,
              ys_ref, w1c_ref, w2c_ref):
    t = pl.program_id(0)
    changed = jnp.logical_or(
        t == 0, uexp_s[t] != uexp_s[jnp.maximum(t - 1, 0)])

    @pl.when(jnp.logical_and(changed, uact_s[t] > 0))
    def _cast():
        w1c_ref[...] = w1_ref[0].astype(jnp.bfloat16)
        w2c_ref[...] = w2_ref[0].astype(jnp.bfloat16)

    @pl.when(uact_s[t] > 0)
    def _():
        H = T // 2
        xa = xs_ref[0:H, :].astype(jnp.bfloat16)
        xb = xs_ref[H:T, :].astype(jnp.bfloat16)
        dn = (((1,), (1,)), ((), ()))
        ha = lax.dot_general(xa, w1c_ref[...], dimension_numbers=dn,
                             preferred_element_type=jnp.float32) + b1_ref[0]
        hb = lax.dot_general(xb, w1c_ref[...], dimension_numbers=dn,
                             preferred_element_type=jnp.float32) + b1_ref[0]
        ha = jnp.maximum(ha, 0.0).astype(jnp.bfloat16)
        hb = jnp.maximum(hb, 0.0).astype(jnp.bfloat16)
        ys_ref[0:H, :] = lax.dot_general(
            ha, w2c_ref[...], dimension_numbers=dn,
            preferred_element_type=jnp.float32) + b2_ref[0]
        ys_ref[H:T, :] = lax.dot_general(
            hb, w2c_ref[...], dimension_numbers=dn,
            preferred_element_type=jnp.float32) + b2_ref[0]


def _ffn(xs, W1, b1, W2, b2, uexp, uact):
    grid_spec = pltpu.PrefetchScalarGridSpec(
        num_scalar_prefetch=2,
        grid=(NTP,),
        in_specs=[
            pl.BlockSpec((T, D), lambda t, ue, ua: (t, 0)),
            pl.BlockSpec((1, DFF, D), lambda t, ue, ua: (ue[t], 0, 0)),
            pl.BlockSpec((1, 1, DFF), lambda t, ue, ua: (ue[t], 0, 0)),
            pl.BlockSpec((1, D, DFF), lambda t, ue, ua: (ue[t], 0, 0)),
            pl.BlockSpec((1, 1, D), lambda t, ue, ua: (ue[t], 0, 0)),
        ],
        out_specs=pl.BlockSpec((T, D), lambda t, ue, ua: (t, 0)),
        scratch_shapes=[
            pltpu.VMEM((DFF, D), jnp.bfloat16),
            pltpu.VMEM((D, DFF), jnp.bfloat16),
        ],
    )
    return pl.pallas_call(
        _ffn_body,
        grid_spec=grid_spec,
        out_shape=jax.ShapeDtypeStruct((P, D), jnp.float32),
    )(uexp, uact, xs, W1, b1, W2, b2)


def _ln_body(x_ref, tw_ref, b0_ref, b1_ref, gamma_ref, beta_ref, out_ref):
    w0 = tw_ref[:, 0:1]
    w1 = tw_ref[:, 1:2]
    res = x_ref[...] + w0 * b0_ref[...] + w1 * b1_ref[...]
    mu = jnp.mean(res, axis=1, keepdims=True)
    var = jnp.mean((res - mu) ** 2, axis=1, keepdims=True)
    out_ref[...] = (gamma_ref[...] * (res - mu)
                    * lax.rsqrt(var + 1e-5) + beta_ref[...])


def _ln(x, top_w, buf, gamma2, beta2):
    nt = N // TNL
    return pl.pallas_call(
        _ln_body,
        grid=(nt,),
        in_specs=[
            pl.BlockSpec((TNL, D), lambda t: (t, 0)),
            pl.BlockSpec((TNL, 2), lambda t: (t, 0)),
            pl.BlockSpec((TNL, D), lambda t: (t, 0)),
            pl.BlockSpec((TNL, D), lambda t: (t + N // TNL, 0)),
            pl.BlockSpec((1, D), lambda t: (0, 0)),
            pl.BlockSpec((1, D), lambda t: (0, 0)),
        ],
        out_specs=pl.BlockSpec((TNL, D), lambda t: (t, 0)),
        out_shape=jax.ShapeDtypeStruct((N, D), jnp.float32),
    )(x, top_w, buf, buf, gamma2, beta2)


@jax.jit
def kernel(x, Wg, bg, W1, b1, W2, b2, gamma, beta):
    i32 = jnp.int32
    bg2 = bg.reshape(1, E)
    gamma2 = gamma.reshape(1, D)
    beta2 = beta.reshape(1, D)

    top_idx, top_w = _route(x, Wg, bg2)

    ef = top_idx.reshape(-1)
    oh = (ef[:, None] == jnp.arange(E, dtype=i32)[None, :]).astype(i32)
    ranks = jnp.cumsum(oh, axis=0)
    counts = ranks[-1]
    pcount = ((counts + (T - 1)) // T) * T
    pends = jnp.cumsum(pcount)
    poff = pends - pcount
    rank_a = jnp.sum(oh * ranks, axis=1)
    base_a = jnp.sum(oh * poff[None, :], axis=1)
    dest = (base_a + rank_a - 1).astype(i32)
    tokidx = (jnp.arange(A, dtype=i32) // 2)
    g = dest.reshape(N, 2).T.reshape(-1)

    starts_r = jnp.arange(NTP, dtype=i32) * T
    ue_raw = jnp.searchsorted(pends, starts_r, side="right").astype(i32)
    uact = (starts_r < pends[-1]).astype(i32)
    ue = jnp.maximum(
        lax.cummax(jnp.where(uact > 0, jnp.clip(ue_raw, 0, E - 1), -1)),
        0).astype(i32)

    xs = _sc_reorder(x, tokidx, dest)
    ys = _ffn(xs, W1, b1.reshape(E, 1, DFF), W2, b2.reshape(E, 1, D),
              ue, uact)
    buf = _sc_unsort(ys, g)
    return _ln(x, top_w, buf, gamma2, beta2)


# speedup vs baseline: 1.3102x; 1.0123x over previous
import functools

import jax
import jax.numpy as jnp
from jax import lax
from jax.experimental import pallas as pl
from jax.experimental.pallas import tpu as pltpu
from jax.experimental.pallas import tpu_sc as plsc

D = 768
E = 8
DFF = 2 * D
N = 2048
A = 2 * N
T = 512
P = A + E * T
NTP = P // T

NC = 2
NS = 16
NW = NC * NS
BPW = A // NW

TNL = 512


def _route_body(x_ref, wg_ref, bg_ref, idx_ref, w_ref):
    logits = lax.dot_general(
        x_ref[...], wg_ref[...],
        dimension_numbers=(((1,), (1,)), ((), ())),
        preferred_element_type=jnp.float32,
    ) + bg_ref[...]
    ii = lax.broadcasted_iota(jnp.int32, (N, E), 1)
    v1 = jnp.max(logits, axis=1, keepdims=True)
    i1 = jnp.min(jnp.where(logits == v1, ii, E), axis=1, keepdims=True)
    m1 = ii == i1
    neg = jnp.where(m1, -jnp.inf, logits)
    v2 = jnp.max(neg, axis=1, keepdims=True)
    i2 = jnp.min(jnp.where(neg == v2, ii, E), axis=1, keepdims=True)
    z = jnp.exp(v2 - v1)
    sm1 = 1.0 / (1.0 + z)
    sm2 = z / (1.0 + z)
    idx_ref[...] = jnp.concatenate([i1, i2], axis=1)
    w_ref[...] = jnp.concatenate([sm1, sm2], axis=1)


def _route(x, Wg, bg2):
    return pl.pallas_call(
        _route_body,
        out_shape=(jax.ShapeDtypeStruct((N, 2), jnp.int32),
                   jax.ShapeDtypeStruct((N, 2), jnp.float32)),
    )(x, Wg, bg2)


def _sc_reorder(x, tokidx, dest):
    @functools.partial(
        pl.kernel,
        out_type=jax.ShapeDtypeStruct((P, D), jnp.float32),
        mesh=plsc.VectorSubcoreMesh(core_axis_name="c", subcore_axis_name="s"),
        scratch_types=[
            pltpu.VMEM((BPW,), jnp.int32),
            pltpu.VMEM((BPW,), jnp.int32),
            pltpu.VMEM((BPW, D), jnp.float32),
            pltpu.SemaphoreType.DMA,
        ],
    )
    def k(x_hbm, tok_hbm, dst_hbm, out_hbm, tok_v, dst_v, rows_v, sem):
        wid = lax.axis_index("s") * NC + lax.axis_index("c")
        base = wid * BPW
        pltpu.sync_copy(tok_hbm.at[pl.ds(base, BPW)], tok_v)
        pltpu.sync_copy(dst_hbm.at[pl.ds(base, BPW)], dst_v)
        pltpu.async_copy(x_hbm.at[tok_v], rows_v, sem).wait()
        pltpu.async_copy(rows_v, out_hbm.at[dst_v], sem).wait()

    return k(x, tokidx, dest)


def _sc_unsort(ys, g):
    @functools.partial(
        pl.kernel,
        out_type=jax.ShapeDtypeStruct((A, D), jnp.float32),
        mesh=plsc.VectorSubcoreMesh(core_axis_name="c", subcore_axis_name="s"),
        scratch_types=[
            pltpu.VMEM((BPW,), jnp.int32),
            pltpu.VMEM((BPW, D), jnp.float32),
            pltpu.SemaphoreType.DMA,
        ],
    )
    def k(ys_hbm, g_hbm, out_hbm, idx_v, rows_v, sem):
        wid = lax.axis_index("s") * NC + lax.axis_index("c")
        base = wid * BPW
        pltpu.sync_copy(g_hbm.at[pl.ds(base, BPW)], idx_v)
        pltpu.async_copy(ys_hbm.at[idx_v], rows_v, sem).wait()
        pltpu.sync_copy(rows_v, out_hbm.at[pl.ds(base, BPW)])

    return k(ys, g)


def _ffn_body(uexp_s, uact_s, xs_ref, w1_ref, b1_ref, w2_ref, b2_ref,
              ys_ref, w1c_ref, w2c_ref):
    t = pl.program_id(0)
    changed = jnp.logical_or(
        t == 0, uexp_s[t] != uexp_s[jnp.maximum(t - 1, 0)])

    @pl.when(jnp.logical_and(changed, uact_s[t] > 0))
    def _cast():
        w1c_ref[...] = w1_ref[0].astype(jnp.bfloat16)
        w2c_ref[...] = w2_ref[0].astype(jnp.bfloat16)

    @pl.when(uact_s[t] > 0)
    def _():
        H = T // 2
        xa = xs_ref[0:H, :].astype(jnp.bfloat16)
        xb = xs_ref[H:T, :].astype(jnp.bfloat16)
        dn = (((1,), (1,)), ((), ()))
        ha = lax.dot_general(xa, w1c_ref[...], dimension_numbers=dn,
                             preferred_element_type=jnp.float32) + b1_ref[0]
        hb = lax.dot_general(xb, w1c_ref[...], dimension_numbers=dn,
                             preferred_element_type=jnp.float32) + b1_ref[0]
        ha = jnp.maximum(ha, 0.0).astype(jnp.bfloat16)
        hb = jnp.maximum(hb, 0.0).astype(jnp.bfloat16)
        ys_ref[0:H, :] = lax.dot_general(
            ha, w2c_ref[...], dimension_numbers=dn,
            preferred_element_type=jnp.float32) + b2_ref[0]
        ys_ref[H:T, :] = lax.dot_general(
            hb, w2c_ref[...], dimension_numbers=dn,
            preferred_element_type=jnp.float32) + b2_ref[0]


def _ffn(xs, W1, b1, W2, b2, uexp, uact):
    grid_spec = pltpu.PrefetchScalarGridSpec(
        num_scalar_prefetch=2,
        grid=(NTP,),
        in_specs=[
            pl.BlockSpec((T, D), lambda t, ue, ua: (t, 0)),
            pl.BlockSpec((1, DFF, D), lambda t, ue, ua: (ue[t], 0, 0)),
            pl.BlockSpec((1, 1, DFF), lambda t, ue, ua: (ue[t], 0, 0)),
            pl.BlockSpec((1, D, DFF), lambda t, ue, ua: (ue[t], 0, 0)),
            pl.BlockSpec((1, 1, D), lambda t, ue, ua: (ue[t], 0, 0)),
        ],
        out_specs=pl.BlockSpec((T, D), lambda t, ue, ua: (t, 0)),
        scratch_shapes=[
            pltpu.VMEM((DFF, D), jnp.bfloat16),
            pltpu.VMEM((D, DFF), jnp.bfloat16),
        ],
    )
    return pl.pallas_call(
        _ffn_body,
        grid_spec=grid_spec,
        out_shape=jax.ShapeDtypeStruct((P, D), jnp.float32),
    )(uexp, uact, xs, W1, b1, W2, b2)


def _ln_body(x_ref, tw_ref, b0_ref, b1_ref, gamma_ref, beta_ref, out_ref):
    w0 = tw_ref[:, 0:1]
    w1 = tw_ref[:, 1:2]
    res = x_ref[...] + w0 * b0_ref[...] + w1 * b1_ref[...]
    mu = jnp.mean(res, axis=1, keepdims=True)
    var = jnp.mean((res - mu) ** 2, axis=1, keepdims=True)
    out_ref[...] = (gamma_ref[...] * (res - mu)
                    * lax.rsqrt(var + 1e-5) + beta_ref[...])


def _ln(x, top_w, buf, gamma2, beta2):
    nt = N // TNL
    return pl.pallas_call(
        _ln_body,
        grid=(nt,),
        in_specs=[
            pl.BlockSpec((TNL, D), lambda t: (t, 0)),
            pl.BlockSpec((TNL, 2), lambda t: (t, 0)),
            pl.BlockSpec((TNL, D), lambda t: (t, 0)),
            pl.BlockSpec((TNL, D), lambda t: (t + N // TNL, 0)),
            pl.BlockSpec((1, D), lambda t: (0, 0)),
            pl.BlockSpec((1, D), lambda t: (0, 0)),
        ],
        out_specs=pl.BlockSpec((TNL, D), lambda t: (t, 0)),
        out_shape=jax.ShapeDtypeStruct((N, D), jnp.float32),
    )(x, top_w, buf, buf, gamma2, beta2)


@jax.jit
def kernel(x, Wg, bg, W1, b1, W2, b2, gamma, beta):
    i32 = jnp.int32
    bg2 = bg.reshape(1, E)
    gamma2 = gamma.reshape(1, D)
    beta2 = beta.reshape(1, D)

    top_idx, top_w = _route(x, Wg, bg2)

    ef = top_idx.reshape(-1)
    oh = (ef[:, None] == jnp.arange(E, dtype=i32)[None, :]).astype(i32)
    ranks = jnp.cumsum(oh, axis=0)
    counts = ranks[-1]
    pcount = ((counts + (T - 1)) // T) * T
    pends = jnp.cumsum(pcount)
    poff = pends - pcount
    rank_a = jnp.sum(oh * ranks, axis=1)
    base_a = jnp.sum(oh * poff[None, :], axis=1)
    dest = (base_a + rank_a - 1).astype(i32)
    tokidx = (jnp.arange(A, dtype=i32) // 2)
    g = dest.reshape(N, 2).T.reshape(-1)

    starts_r = jnp.arange(NTP, dtype=i32) * T
    ue_raw = jnp.searchsorted(pends, starts_r, side="right").astype(i32)
    uact = (starts_r < pends[-1]).astype(i32)
    ue = jnp.maximum(
        lax.cummax(jnp.where(uact > 0, jnp.clip(ue_raw, 0, E - 1), -1)),
        0).astype(i32)

    xs = _sc_reorder(x, tokidx, dest)
    ys = _ffn(xs, W1, b1.reshape(E, 1, DFF), W2, b2.reshape(E, 1, D),
              ue, uact)
    buf = _sc_unsort(ys, g)
    return _ln(x, top_w, buf, gamma2, beta2)
